# baseline (speedup 1.0000x reference)
import functools

import jax
import jax.numpy as jnp
from jax.experimental import pallas as pl
from jax.experimental.pallas import tpu as pltpu


def _lrelu(v):
    return jnp.where(v > 0, v, 0.2 * v)


def _gat_kernel(x_ref, xT_ref, eT_ref,
                Ws1_ref, bs1_ref, Ws2_ref, bs2_ref,
                Wn1_ref, bn1_ref, Wn2_ref, bn2_ref,
                Wn1T_ref, bn1c_ref, Wn2T_ref, bn2c_ref,
                Wc1s_ref, bc1_ref, Wc1nT_ref,
                c06_ref, c04_ref,
                bc2_ref, kpos_ref,
                out_ref,
                tab_col_ref, tab_row_ref):
    n = x_ref.shape[1]
    h = Ws2_ref.shape[1]
    dot = functools.partial(jnp.dot, preferred_element_type=jnp.float32)

    x = x_ref[0]
    xT = xT_ref[0]

    se = dot(_lrelu(dot(x, Ws1_ref[...]) + bs1_ref[...]), Ws2_ref[...]) + bs2_ref[...]
    ne = dot(_lrelu(dot(x, Wn1_ref[...]) + bn1_ref[...]), Wn2_ref[...]) + bn2_ref[...]
    neT = dot(Wn2T_ref[...], _lrelu(dot(Wn1T_ref[...], xT) + bn1c_ref[...])) + bn2c_ref[...]

    ps = dot(se, Wc1s_ref[...]) + bc1_ref[...]
    pnT = dot(Wc1nT_ref[...], neT)

    u06 = dot(ps, c06_ref[...])
    v06 = dot(c06_ref[...].T, pnT)

    a_row = jnp.abs(c04_ref[...]).reshape(1, h)
    a_col = jnp.abs(c04_ref[...])
    psA = ps * a_row
    pnTA = pnT * a_col

    for k in range(h):
        tab_col_ref[k] = jax.lax.slice(psA, (0, k), (n, k + 1))
        tab_row_ref[k] = jax.lax.slice(pnTA, (k, 0), (k + 1, n))

    kpos = kpos_ref[0, 0]

    def body(k, acc):
        return acc + jnp.abs(tab_col_ref[k] + tab_row_ref[k])

    accP = jax.lax.fori_loop(0, kpos, body, jnp.zeros((n, n), jnp.float32))
    accN = jax.lax.fori_loop(kpos, h, body, jnp.zeros((n, n), jnp.float32))
    imp = u06 + v06 + (accP - accN) + bc2_ref[...]

    ii = jax.lax.broadcasted_iota(jnp.int32, (n, n), 0)
    jj = jax.lax.broadcasted_iota(jnp.int32, (n, n), 1)
    mask = (eT_ref[0] != 0) & (ii != jj)
    logits = jnp.where(mask, imp, -1e30)
    m = jnp.max(logits, axis=1, keepdims=True)
    e = jnp.exp(logits - m)
    s = jnp.sum(e, axis=1, keepdims=True)
    w = (e / s) * mask.astype(jnp.float32)

    sum_nb = dot(w, ne)
    has = jnp.max(mask.astype(jnp.float32), axis=1, keepdims=True) > 0
    out_ref[0] = jnp.where(has, sum_nb + se, 0.0)


def kernel(nodes, edges, Ws1, bs1, Ws2, bs2, Wn1, bn1, Wn2, bn2, Wc1, bc1, Wc2, bc2):
    b, n = nodes.shape[0], nodes.shape[1]
    d = nodes.shape[2] * nodes.shape[3]
    h = Ws2.shape[1]

    x = nodes.reshape(b, n, d)
    xT = jnp.swapaxes(x, 1, 2)
    eT = jnp.swapaxes(edges, 1, 2)

    perm = jnp.argsort((Wc2[:, 0] < 0).astype(jnp.int32), stable=True)
    Wc1p = Wc1[:, perm]
    bc1p = bc1[perm]
    Wc2p = Wc2[perm]
    kpos = jnp.sum((Wc2p[:, 0] >= 0).astype(jnp.int32)).reshape(1, 1)

    row = lambda v: v.reshape(1, -1)
    col = lambda v: v.reshape(-1, 1)

    operands = (
        x, xT, eT,
        Ws1, row(bs1), Ws2, row(bs2),
        Wn1, row(bn1), Wn2, row(bn2),
        Wn1.T, col(bn1), Wn2.T, col(bn2),
        Wc1p[:h], row(bc1p), Wc1p[h:].T,
        0.6 * Wc2p, 0.4 * Wc2p,
        bc2.reshape(1, 1), kpos,
    )

    def bspec(a):
        if a.ndim == 3:
            return pl.BlockSpec((1,) + a.shape[1:], lambda i: (i, 0, 0))
        return pl.BlockSpec(a.shape, lambda i: (0,) * a.ndim)

    return pl.pallas_call(
        _gat_kernel,
        grid=(b,),
        in_specs=[bspec(a) for a in operands],
        out_specs=pl.BlockSpec((1, n, h), lambda i: (i, 0, 0)),
        out_shape=jax.ShapeDtypeStruct((b, n, h), jnp.float32),
        scratch_shapes=[
            pltpu.VMEM((h, n, 1), jnp.float32),
            pltpu.VMEM((h, 1, n), jnp.float32),
        ],
    )(*operands)


# speedup vs baseline: 2.1502x; 2.1502x over previous
import functools

import jax
import jax.numpy as jnp
from jax.experimental import pallas as pl


def _lrelu(v):
    return jnp.where(v > 0, v, 0.2 * v)


def _gat_kernel(x_ref, e_ref,
                Ws1_ref, bs1_ref, Ws2_ref, bs2_ref,
                Wn1_ref, bn1_ref, Wn2_ref, bn2_ref,
                Wc1s_ref, bc1_ref, Wc1n_ref,
                c06_ref, c04_ref, bc2_ref,
                out_ref):
    n = x_ref.shape[1]
    h = Ws2_ref.shape[1]
    dot = functools.partial(jnp.dot, preferred_element_type=jnp.float32)

    x = x_ref[0]

    se = dot(_lrelu(dot(x, Ws1_ref[...]) + bs1_ref[...]), Ws2_ref[...]) + bs2_ref[...]
    ne = dot(_lrelu(dot(x, Wn1_ref[...]) + bn1_ref[...]), Wn2_ref[...]) + bn2_ref[...]

    ps = dot(se, Wc1s_ref[...]) + bc1_ref[...]
    pn = dot(ne, Wc1n_ref[...])
    psT = jnp.swapaxes(ps, 0, 1)

    u06 = dot(c06_ref[...].T, psT)
    v06 = dot(pn, c06_ref[...])
    c04 = c04_ref[...]

    acc = jnp.zeros((n, n), jnp.float32)
    for k in range(h):
        pn_k = jax.lax.slice(pn, (0, k), (n, k + 1))
        ps_k = jax.lax.slice(psT, (k, 0), (k + 1, n))
        c_k = jax.lax.slice(c04, (k, 0), (k + 1, 1))
        acc = acc + c_k * jnp.abs(pn_k + ps_k)
    impT = u06 + v06 + acc + bc2_ref[...]

    ii = jax.lax.broadcasted_iota(jnp.int32, (n, n), 0)
    jj = jax.lax.broadcasted_iota(jnp.int32, (n, n), 1)
    maskT = (e_ref[0] != 0) & (ii != jj)
    logitsT = jnp.where(maskT, impT, -1e30)
    m = jnp.max(logitsT, axis=0, keepdims=True)
    e = jnp.exp(logitsT - m)
    s = jnp.sum(e, axis=0, keepdims=True)
    wT = (e / s) * maskT.astype(jnp.float32)

    neT = jnp.swapaxes(ne, 0, 1)
    sum_nbT = dot(neT, wT)
    seT = jnp.swapaxes(se, 0, 1)
    hasT = jnp.max(maskT.astype(jnp.float32), axis=0, keepdims=True) > 0
    outT = jnp.where(hasT, sum_nbT + seT, 0.0)
    out_ref[0] = jnp.swapaxes(outT, 0, 1)


def kernel(nodes, edges, Ws1, bs1, Ws2, bs2, Wn1, bn1, Wn2, bn2, Wc1, bc1, Wc2, bc2):
    b, n = nodes.shape[0], nodes.shape[1]
    d = nodes.shape[2] * nodes.shape[3]
    h = Ws2.shape[1]

    x = nodes.reshape(b, n, d)

    row = lambda v: v.reshape(1, -1)

    operands = (
        x, edges,
        Ws1, row(bs1), Ws2, row(bs2),
        Wn1, row(bn1), Wn2, row(bn2),
        Wc1[:h], row(bc1), Wc1[h:],
        0.6 * Wc2, 0.4 * Wc2, bc2.reshape(1, 1),
    )

    def bspec(a):
        if a.ndim == 3:
            return pl.BlockSpec((1,) + a.shape[1:], lambda i: (i, 0, 0))
        return pl.BlockSpec(a.shape, lambda i: (0,) * a.ndim)

    return pl.pallas_call(
        _gat_kernel,
        grid=(b,),
        in_specs=[bspec(a) for a in operands],
        out_specs=pl.BlockSpec((1, n, h), lambda i: (i, 0, 0)),
        out_shape=jax.ShapeDtypeStruct((b, n, h), jnp.float32),
    )(*operands)


# speedup vs baseline: 2.6874x; 1.2498x over previous
import functools

import jax
import jax.numpy as jnp
from jax.experimental import pallas as pl


def _lrelu(v):
    return jnp.where(v > 0, v, 0.2 * v)


def _gat_kernel(xT_ref, e_ref,
                Ws1T_ref, bs1_ref, Ws2_ref, bs2_ref,
                Wn1T_ref, bn1_ref, Wn2_ref, bn2_ref,
                Wc1T_ref, bc1_ref, Wc2_ref, bc2_ref,
                out_ref):
    n = xT_ref.shape[2]
    h = Ws2_ref.shape[1]
    dot = functools.partial(jnp.dot, preferred_element_type=jnp.float32)

    xT = xT_ref[0]
    col = lambda r: jnp.swapaxes(r, 0, 1)

    bs1c, bs2c = col(bs1_ref[...]), col(bs2_ref[...])
    bn1c, bn2c = col(bn1_ref[...]), col(bn2_ref[...])
    bc1c = col(bc1_ref[...])

    seT = dot(Ws2_ref[...].T, _lrelu(dot(Ws1T_ref[...], xT) + bs1c)) + bs2c
    neT = dot(Wn2_ref[...].T, _lrelu(dot(Wn1T_ref[...], xT) + bn1c)) + bn2c

    Wc1sT = Wc1T_ref[...][:, :h]
    Wc1nT = Wc1T_ref[...][:, h:]
    psT = dot(Wc1sT, seT) + bc1c
    pnT = dot(Wc1nT, neT)
    pn = jnp.swapaxes(pnT, 0, 1)

    c = Wc2_ref[...]
    c06 = 0.6 * c
    c04 = 0.4 * c
    u06 = dot(c06.T, psT)
    v06 = dot(pn, c06)

    acc = jnp.zeros((n, n), jnp.float32)
    for k in range(h):
        pn_k = jax.lax.slice(pn, (0, k), (n, k + 1))
        ps_k = jax.lax.slice(psT, (k, 0), (k + 1, n))
        c_k = jax.lax.slice(c04, (k, 0), (k + 1, 1))
        acc = acc + c_k * jnp.abs(pn_k + ps_k)
    impT = u06 + v06 + acc + bc2_ref[...]

    ii = jax.lax.broadcasted_iota(jnp.int32, (n, n), 0)
    jj = jax.lax.broadcasted_iota(jnp.int32, (n, n), 1)
    maskT = (e_ref[0] != 0) & (ii != jj)
    logitsT = jnp.where(maskT, impT, -1e30)
    m = jnp.max(logitsT, axis=0, keepdims=True)
    e = jnp.exp(logitsT - m)
    s = jnp.sum(e, axis=0, keepdims=True)
    wT = (e / s) * maskT.astype(jnp.float32)

    sum_nbT = dot(neT, wT)
    hasT = jnp.max(maskT.astype(jnp.float32), axis=0, keepdims=True) > 0
    out_ref[0] = jnp.where(hasT, sum_nbT + seT, 0.0)


def kernel(nodes, edges, Ws1, bs1, Ws2, bs2, Wn1, bn1, Wn2, bn2, Wc1, bc1, Wc2, bc2):
    b, n = nodes.shape[0], nodes.shape[1]
    d = nodes.shape[2] * nodes.shape[3]
    h = Ws2.shape[1]

    xT = jnp.swapaxes(nodes.reshape(b, n, d), 1, 2)

    row = lambda v: v.reshape(1, -1)

    operands = (
        xT, edges,
        Ws1.T, row(bs1), Ws2, row(bs2),
        Wn1.T, row(bn1), Wn2, row(bn2),
        Wc1.T, row(bc1), Wc2, bc2.reshape(1, 1),
    )

    def bspec(a):
        if a.ndim == 3:
            return pl.BlockSpec((1,) + a.shape[1:], lambda i: (i, 0, 0))
        return pl.BlockSpec(a.shape, lambda i: (0,) * a.ndim)

    outT = pl.pallas_call(
        _gat_kernel,
        grid=(b,),
        in_specs=[bspec(a) for a in operands],
        out_specs=pl.BlockSpec((1, h, n), lambda i: (i, 0, 0)),
        out_shape=jax.ShapeDtypeStruct((b, h, n), jnp.float32),
    )(*operands)
    return jnp.swapaxes(outT, 1, 2)


# speedup vs baseline: 3.9913x; 1.4852x over previous
import functools

import jax
import jax.numpy as jnp
from jax.experimental import pallas as pl


def _lrelu(v):
    return jnp.where(v > 0, v, 0.2 * v)


def _gat_kernel(xT_ref, e_ref,
                Ws1T_ref, bs1_ref, Ws2_ref, bs2_ref,
                Wn1T_ref, bn1_ref, Wn2_ref, bn2_ref,
                Wc1T_ref, bc1_ref, Wc2_ref, bc2_ref,
                out_ref):
    n = xT_ref.shape[2]
    h = Ws2_ref.shape[1]
    dot = functools.partial(jnp.dot, preferred_element_type=jnp.float32)

    xT = xT_ref[0]
    col = lambda r: jnp.swapaxes(r, 0, 1)

    bs1c, bs2c = col(bs1_ref[...]), col(bs2_ref[...])
    bn1c, bn2c = col(bn1_ref[...]), col(bn2_ref[...])
    bc1c = col(bc1_ref[...])

    seT = dot(Ws2_ref[...].T, _lrelu(dot(Ws1T_ref[...], xT) + bs1c)) + bs2c
    neT = dot(Wn2_ref[...].T, _lrelu(dot(Wn1T_ref[...], xT) + bn1c)) + bn2c

    Wc1sT = Wc1T_ref[...][:, :h]
    Wc1nT = Wc1T_ref[...][:, h:]
    psT = dot(Wc1sT, seT) + bc1c
    pnT = dot(Wc1nT, neT)
    pn = jnp.swapaxes(pnT, 0, 1)

    c = Wc2_ref[...]
    c06 = 0.6 * c
    c04 = 0.4 * c
    u06 = dot(c06.T, psT)
    v06 = dot(pn, c06)

    pnb = pn.astype(jnp.bfloat16)
    psTb = psT.astype(jnp.bfloat16)
    c04b = c04.astype(jnp.bfloat16)
    GRP = 8
    acc = jnp.zeros((n, n), jnp.float32)
    for g in range(0, h, GRP):
        accg = jnp.zeros((n, n), jnp.bfloat16)
        for k in range(g, g + GRP):
            pn_k = jax.lax.slice(pnb, (0, k), (n, k + 1))
            ps_k = jax.lax.slice(psTb, (k, 0), (k + 1, n))
            c_k = jax.lax.slice(c04b, (k, 0), (k + 1, 1))
            accg = accg + c_k * jnp.abs(pn_k + ps_k)
        acc = acc + accg.astype(jnp.float32)
    impT = u06 + v06 + acc + bc2_ref[...]

    ii = jax.lax.broadcasted_iota(jnp.int32, (n, n), 0)
    jj = jax.lax.broadcasted_iota(jnp.int32, (n, n), 1)
    maskT = (e_ref[0] != 0) & (ii != jj)
    logitsT = jnp.where(maskT, impT, -1e30)
    m = jnp.max(logitsT, axis=0, keepdims=True)
    e = jnp.exp(logitsT - m)
    s = jnp.sum(e, axis=0, keepdims=True)
    wT = (e / s) * maskT.astype(jnp.float32)

    sum_nbT = dot(neT, wT)
    hasT = jnp.max(maskT.astype(jnp.float32), axis=0, keepdims=True) > 0
    out_ref[0] = jnp.where(hasT, sum_nbT + seT, 0.0)


def kernel(nodes, edges, Ws1, bs1, Ws2, bs2, Wn1, bn1, Wn2, bn2, Wc1, bc1, Wc2, bc2):
    b, n = nodes.shape[0], nodes.shape[1]
    d = nodes.shape[2] * nodes.shape[3]
    h = Ws2.shape[1]

    xT = jnp.swapaxes(nodes.reshape(b, n, d), 1, 2)

    row = lambda v: v.reshape(1, -1)

    operands = (
        xT, edges,
        Ws1.T, row(bs1), Ws2, row(bs2),
        Wn1.T, row(bn1), Wn2, row(bn2),
        Wc1.T, row(bc1), Wc2, bc2.reshape(1, 1),
    )

    def bspec(a):
        if a.ndim == 3:
            return pl.BlockSpec((1,) + a.shape[1:], lambda i: (i, 0, 0))
        return pl.BlockSpec(a.shape, lambda i: (0,) * a.ndim)

    outT = pl.pallas_call(
        _gat_kernel,
        grid=(b,),
        in_specs=[bspec(a) for a in operands],
        out_specs=pl.BlockSpec((1, h, n), lambda i: (i, 0, 0)),
        out_shape=jax.ShapeDtypeStruct((b, h, n), jnp.float32),
    )(*operands)
    return jnp.swapaxes(outT, 1, 2)


# speedup vs baseline: 4.1816x; 1.0477x over previous
import functools

import jax
import jax.numpy as jnp
from jax.experimental import pallas as pl


def _lrelu(v):
    return jnp.where(v > 0, v, 0.2 * v)


def _gat_kernel(xT_ref, e_ref,
                Ws1T_ref, bs1_ref, Ws2_ref, bs2_ref,
                Wn1T_ref, bn1_ref, Wn2_ref, bn2_ref,
                Wc1T_ref, bc1_ref, Wc2_ref, bc2_ref,
                out_ref):
    n = xT_ref.shape[2]
    h = Ws2_ref.shape[1]
    dot = functools.partial(jnp.dot, preferred_element_type=jnp.float32)

    xT = xT_ref[0]
    col = lambda r: jnp.swapaxes(r, 0, 1)

    bs1c, bs2c = col(bs1_ref[...]), col(bs2_ref[...])
    bn1c, bn2c = col(bn1_ref[...]), col(bn2_ref[...])
    bc1c = col(bc1_ref[...])

    seT = dot(Ws2_ref[...].T, _lrelu(dot(Ws1T_ref[...], xT) + bs1c)) + bs2c
    neT = dot(Wn2_ref[...].T, _lrelu(dot(Wn1T_ref[...], xT) + bn1c)) + bn2c

    Wc1sT = Wc1T_ref[...][:, :h]
    Wc1nT = Wc1T_ref[...][:, h:]
    psT = dot(Wc1sT, seT) + bc1c
    pnT = dot(Wc1nT, neT)
    pn = jnp.swapaxes(pnT, 0, 1)

    c = Wc2_ref[...]
    c06 = 0.6 * c
    c04 = 0.4 * c
    u06 = dot(c06.T, psT)
    v06 = dot(pn, c06)

    pnb = pn.astype(jnp.bfloat16)
    psTb = psT.astype(jnp.bfloat16)
    c04b = c04.astype(jnp.bfloat16)
    GRP = 16
    acc = jnp.zeros((n, n), jnp.float32)
    for g in range(0, h, GRP):
        accg = jnp.zeros((n, n), jnp.bfloat16)
        for k in range(g, g + GRP):
            pn_k = jax.lax.slice(pnb, (0, k), (n, k + 1))
            ps_k = jax.lax.slice(psTb, (k, 0), (k + 1, n))
            c_k = jax.lax.slice(c04b, (k, 0), (k + 1, 1))
            accg = accg + c_k * jnp.abs(pn_k + ps_k)
        acc = acc + accg.astype(jnp.float32)
    impT = u06 + v06 + acc + bc2_ref[...]

    ii = jax.lax.broadcasted_iota(jnp.int32, (n, n), 0)
    jj = jax.lax.broadcasted_iota(jnp.int32, (n, n), 1)
    maskT = (e_ref[0] != 0) & (ii != jj)
    logitsT = jnp.where(maskT, impT, -1e30)
    m = jnp.max(logitsT, axis=0, keepdims=True)
    e = jnp.exp(logitsT - m)
    s = jnp.sum(e, axis=0, keepdims=True)
    wT = e / s

    sum_nbT = dot(neT, wT)
    hasT = m > -1e29
    out_ref[0] = jnp.where(hasT, sum_nbT + seT, 0.0)


def kernel(nodes, edges, Ws1, bs1, Ws2, bs2, Wn1, bn1, Wn2, bn2, Wc1, bc1, Wc2, bc2):
    b, n = nodes.shape[0], nodes.shape[1]
    d = nodes.shape[2] * nodes.shape[3]
    h = Ws2.shape[1]

    xT = jnp.swapaxes(nodes.reshape(b, n, d), 1, 2)

    row = lambda v: v.reshape(1, -1)

    operands = (
        xT, edges,
        Ws1.T, row(bs1), Ws2, row(bs2),
        Wn1.T, row(bn1), Wn2, row(bn2),
        Wc1.T, row(bc1), Wc2, bc2.reshape(1, 1),
    )

    def bspec(a):
        if a.ndim == 3:
            return pl.BlockSpec((1,) + a.shape[1:], lambda i: (i, 0, 0))
        return pl.BlockSpec(a.shape, lambda i: (0,) * a.ndim)

    outT = pl.pallas_call(
        _gat_kernel,
        grid=(b,),
        in_specs=[bspec(a) for a in operands],
        out_specs=pl.BlockSpec((1, h, n), lambda i: (i, 0, 0)),
        out_shape=jax.ShapeDtypeStruct((b, h, n), jnp.float32),
    )(*operands)
    return jnp.swapaxes(outT, 1, 2)


# speedup vs baseline: 4.2465x; 1.0155x over previous
import functools

import jax
import jax.numpy as jnp
from jax.experimental import pallas as pl


def _lrelu(v):
    return jnp.where(v > 0, v, 0.2 * v)


def _gat_kernel(xT_ref, e_ref,
                Ws1T_ref, bs1_ref, Ws2_ref, bs2_ref,
                Wn1T_ref, bn1_ref, Wn2_ref, bn2_ref,
                Wc1T_ref, bc1_ref, Wc2_ref, bc2_ref,
                out_ref):
    n = xT_ref.shape[2]
    h = Ws2_ref.shape[1]
    dot = functools.partial(jnp.dot, preferred_element_type=jnp.float32)

    xT = xT_ref[0]
    col = lambda r: jnp.swapaxes(r, 0, 1)

    bs1c, bs2c = col(bs1_ref[...]), col(bs2_ref[...])
    bn1c, bn2c = col(bn1_ref[...]), col(bn2_ref[...])
    bc1c = col(bc1_ref[...])

    seT = dot(Ws2_ref[...].T, _lrelu(dot(Ws1T_ref[...], xT) + bs1c)) + bs2c
    neT = dot(Wn2_ref[...].T, _lrelu(dot(Wn1T_ref[...], xT) + bn1c)) + bn2c

    Wc1sT = Wc1T_ref[...][:, :h]
    Wc1nT = Wc1T_ref[...][:, h:]
    psT = dot(Wc1sT, seT) + bc1c
    pnT = dot(Wc1nT, neT)
    pn = jnp.swapaxes(pnT, 0, 1)

    c = Wc2_ref[...]
    c06 = 0.6 * c
    c04 = 0.4 * c
    u06 = dot(c06.T, psT)
    v06 = dot(pn, c06)

    pnb = pn.astype(jnp.bfloat16)
    psTb = psT.astype(jnp.bfloat16)
    c04b = c04.astype(jnp.bfloat16)
    GRP = 32
    acc = jnp.zeros((n, n), jnp.float32)
    for g in range(0, h, GRP):
        accg = jnp.zeros((n, n), jnp.bfloat16)
        for k in range(g, g + GRP):
            pn_k = jax.lax.slice(pnb, (0, k), (n, k + 1))
            ps_k = jax.lax.slice(psTb, (k, 0), (k + 1, n))
            c_k = jax.lax.slice(c04b, (k, 0), (k + 1, 1))
            accg = accg + c_k * jnp.abs(pn_k + ps_k)
        acc = acc + accg.astype(jnp.float32)
    impT = u06 + v06 + acc + bc2_ref[...]

    ii = jax.lax.broadcasted_iota(jnp.int32, (n, n), 0)
    jj = jax.lax.broadcasted_iota(jnp.int32, (n, n), 1)
    maskT = (e_ref[0] != 0) & (ii != jj)
    logitsT = jnp.where(maskT, impT, -1e30)
    m = jnp.max(logitsT, axis=0, keepdims=True)
    e = jnp.exp(logitsT - m)
    s = jnp.sum(e, axis=0, keepdims=True)
    wT = e / s

    sum_nbT = dot(neT, wT)
    hasT = m > -1e29
    out_ref[0] = jnp.where(hasT, sum_nbT + seT, 0.0)


def kernel(nodes, edges, Ws1, bs1, Ws2, bs2, Wn1, bn1, Wn2, bn2, Wc1, bc1, Wc2, bc2):
    b, n = nodes.shape[0], nodes.shape[1]
    d = nodes.shape[2] * nodes.shape[3]
    h = Ws2.shape[1]

    xT = jnp.swapaxes(nodes.reshape(b, n, d), 1, 2)

    row = lambda v: v.reshape(1, -1)

    operands = (
        xT, edges,
        Ws1.T, row(bs1), Ws2, row(bs2),
        Wn1.T, row(bn1), Wn2, row(bn2),
        Wc1.T, row(bc1), Wc2, bc2.reshape(1, 1),
    )

    def bspec(a):
        if a.ndim == 3:
            return pl.BlockSpec((1,) + a.shape[1:], lambda i: (i, 0, 0))
        return pl.BlockSpec(a.shape, lambda i: (0,) * a.ndim)

    outT = pl.pallas_call(
        _gat_kernel,
        grid=(b,),
        in_specs=[bspec(a) for a in operands],
        out_specs=pl.BlockSpec((1, h, n), lambda i: (i, 0, 0)),
        out_shape=jax.ShapeDtypeStruct((b, h, n), jnp.float32),
    )(*operands)
    return jnp.swapaxes(outT, 1, 2)


# speedup vs baseline: 4.3514x; 1.0247x over previous
import functools

import jax
import jax.numpy as jnp
from jax.experimental import pallas as pl


def _lrelu(v):
    return jnp.where(v > 0, v, 0.2 * v)


def _gat_kernel(xT_ref, e_ref,
                Ws1T_ref, bs1_ref, Ws2_ref, bs2_ref,
                Wn1T_ref, bn1_ref, Wn2_ref, bn2_ref,
                Wc1T_ref, bc1_ref, Wc2_ref, bc2_ref,
                out_ref):
    n = xT_ref.shape[2]
    h = Ws2_ref.shape[1]
    dot = functools.partial(jnp.dot, preferred_element_type=jnp.float32)

    xT = xT_ref[0]
    col = lambda r: jnp.swapaxes(r, 0, 1)

    bs1c, bs2c = col(bs1_ref[...]), col(bs2_ref[...])
    bn1c, bn2c = col(bn1_ref[...]), col(bn2_ref[...])
    bc1c = col(bc1_ref[...])

    seT = dot(Ws2_ref[...].T, _lrelu(dot(Ws1T_ref[...], xT) + bs1c)) + bs2c
    neT = dot(Wn2_ref[...].T, _lrelu(dot(Wn1T_ref[...], xT) + bn1c)) + bn2c

    Wc1sT = Wc1T_ref[...][:, :h]
    Wc1nT = Wc1T_ref[...][:, h:]
    psT = dot(Wc1sT, seT) + bc1c
    pnT = dot(Wc1nT, neT)
    pn = jnp.swapaxes(pnT, 0, 1)

    c = Wc2_ref[...]
    c06 = 0.6 * c
    c04 = 0.4 * c
    u06 = dot(c06.T, psT)
    v06 = dot(pn, c06)

    pnb = pn.astype(jnp.bfloat16)
    psTb = psT.astype(jnp.bfloat16)
    c04b = c04.astype(jnp.bfloat16)
    GRP = 64
    acc = jnp.zeros((n, n), jnp.float32)
    for g in range(0, h, GRP):
        accg = jnp.zeros((n, n), jnp.bfloat16)
        for k in range(g, g + GRP):
            pn_k = jax.lax.slice(pnb, (0, k), (n, k + 1))
            ps_k = jax.lax.slice(psTb, (k, 0), (k + 1, n))
            c_k = jax.lax.slice(c04b, (k, 0), (k + 1, 1))
            accg = accg + c_k * jnp.abs(pn_k + ps_k)
        acc = acc + accg.astype(jnp.float32)
    impT = u06 + v06 + acc + bc2_ref[...]

    ii = jax.lax.broadcasted_iota(jnp.int32, (n, n), 0)
    jj = jax.lax.broadcasted_iota(jnp.int32, (n, n), 1)
    maskT = (e_ref[0] != 0) & (ii != jj)
    logitsT = jnp.where(maskT, impT, -1e30)
    m = jnp.max(logitsT, axis=0, keepdims=True)
    e = jnp.exp(logitsT - m)
    s = jnp.sum(e, axis=0, keepdims=True)
    wT = e / s

    sum_nbT = dot(neT, wT)
    hasT = m > -1e29
    out_ref[0] = jnp.where(hasT, sum_nbT + seT, 0.0)


def kernel(nodes, edges, Ws1, bs1, Ws2, bs2, Wn1, bn1, Wn2, bn2, Wc1, bc1, Wc2, bc2):
    b, n = nodes.shape[0], nodes.shape[1]
    d = nodes.shape[2] * nodes.shape[3]
    h = Ws2.shape[1]

    xT = jnp.swapaxes(nodes.reshape(b, n, d), 1, 2)

    row = lambda v: v.reshape(1, -1)

    operands = (
        xT, edges,
        Ws1.T, row(bs1), Ws2, row(bs2),
        Wn1.T, row(bn1), Wn2, row(bn2),
        Wc1.T, row(bc1), Wc2, bc2.reshape(1, 1),
    )

    def bspec(a):
        if a.ndim == 3:
            return pl.BlockSpec((1,) + a.shape[1:], lambda i: (i, 0, 0))
        return pl.BlockSpec(a.shape, lambda i: (0,) * a.ndim)

    outT = pl.pallas_call(
        _gat_kernel,
        grid=(b,),
        in_specs=[bspec(a) for a in operands],
        out_specs=pl.BlockSpec((1, h, n), lambda i: (i, 0, 0)),
        out_shape=jax.ShapeDtypeStruct((b, h, n), jnp.float32),
    )(*operands)
    return jnp.swapaxes(outT, 1, 2)
